# baseline (speedup 1.0000x reference)
import functools

import jax
import jax.numpy as jnp
from jax import lax
from jax.experimental import pallas as pl
from jax.experimental.pallas import tpu as pltpu
from jax.experimental.pallas import tpu_sc as plsc

N = 10000
E = 320000
D = 128
B = 64
NC, NS, L = 2, 16, 16
NW = NC * NS
PER = 313
NPAD = NW * PER
PERP = 320
SCAN = 2000
GROUPS = SCAN // L
NCHUNK = E // SCAN
CAP = 4096
FLUSH = CAP - SCAN
PB = 128
UNR = 5
GG = 128
NEG = float("-inf")


def _sc_body(nid_h, e3_h, gid_h, emb_h, ew_h, out_h,
             nid_v, agg0, agg1, agg2, agg3, agg4, agg5, agg6, agg7,
             e3b_v, cl_v, cs_v, ce_v,
             cnts_v, goff_v, vid_v, hrow_v, wv_v, gid_v, pooled_v,
             sem_c0, sem_c1, sem_g0, sem_g1):
    aggs = (agg0, agg1, agg2, agg3, agg4, agg5, agg6, agg7)
    wid = lax.axis_index("c") * NS + lax.axis_index("s")
    n0 = wid * PER

    pltpu.sync_copy(nid_h, nid_v)
    pltpu.sync_copy(gid_h.at[pl.ds(wid * PERP, PERP)], gid_v.at[pl.ds(0, PERP)])

    iota16 = jnp.arange(L, dtype=jnp.int32)
    iota3 = iota16 * 3
    neg_vec = jnp.full((L,), NEG, jnp.float32)
    zero_vec = jnp.zeros((L,), jnp.float32)
    dummy_ivec = jnp.full((L,), PER, jnp.int32)

    def _init(i, _):
        for a in aggs:
            a[pl.ds(i * L, L)] = neg_vec
        return 0
    lax.fori_loop(0, PER + 1, _init, 0)

    def _init2(i, _):
        cl_v[pl.ds(i * L, L)] = dummy_ivec
        cs_v[pl.ds(i * L, L)] = jnp.zeros((L,), jnp.int32)
        ce_v[pl.ds(i * L, L)] = jnp.zeros((L,), jnp.int32)
        return 0
    lax.fori_loop(0, (CAP + L) // L, _init2, 0)

    def _init3(i, _):
        cnts_v[pl.ds(i * L, L)] = jnp.zeros((L,), jnp.int32)
        return 0
    lax.fori_loop(0, GG, _init3, 0)


    def _vids_into(k, slot):
        def _vg(j, _):
            sv = cs_v[pl.ds(k * PB + j * L, L)]
            vid_v[slot, pl.ds(j * L, L)] = plsc.load_gather(nid_v, [sv])
            return 0
        lax.fori_loop(0, PB // L, _vg, 0)

    def _issue_gather(k, slot, sem):
        pltpu.async_copy(emb_h.at[vid_v.at[slot]], hrow_v.at[slot], sem)
        pltpu.async_copy(ew_h.at[ce_v.at[pl.ds(k * PB, PB)]],
                         wv_v.at[slot], sem)

    def _wait_gather(k, slot, sem):
        pltpu.make_async_copy(emb_h.at[vid_v.at[slot]], hrow_v.at[slot],
                              sem).wait()
        pltpu.make_async_copy(ew_h.at[ce_v.at[pl.ds(k * PB, PB)]],
                              wv_v.at[slot], sem).wait()

    def _rmw(k, slot):
        def _r16(j, _):
            locv = cl_v[pl.ds(k * PB + j * L, L)]
            wv16 = wv_v[slot, pl.ds(j * L, L)]
            ei0 = j * L
            for l in range(L):
                lane = jnp.full((L,), l, jnp.int32)
                rb = locv.at[lane].get(mode="promise_in_bounds")
                wb = wv16.at[lane].get(mode="promise_in_bounds")
                addr = rb * L + iota16
                ei = ei0 + l
                for jb in range(D // L):
                    hv = hrow_v[slot, ei, pl.ds(jb * L, L)]
                    av = plsc.load_gather(aggs[jb], [addr])
                    plsc.store_scatter(aggs[jb], [addr],
                                       jnp.maximum(av, hv * wb))
            return 0
        lax.fori_loop(0, PB // L, _r16, 0)

    def _process(cnt):
        nb = (cnt + PB - 1) // PB

        @pl.when(nb > 0)
        def _():
            _vids_into(0, 0)
            _issue_gather(0, 0, sem_g0)

        def _pouter(kk, _):
            for b, sem, osem in ((0, sem_g0, sem_g1), (1, sem_g1, sem_g0)):
                k = kk * 2 + b

                @pl.when(k < nb)
                def _():
                    _wait_gather(k, b, sem)

                    @pl.when(k + 1 < nb)
                    def _():
                        _vids_into(k + 1, 1 - b)
                        _issue_gather(k + 1, 1 - b, osem)

                    _rmw(k, b)
            return 0
        lax.fori_loop(0, (nb + 1) // 2, _pouter, 0)


    def _issue_chunk(g, b, sem):
        pltpu.async_copy(e3_h.at[pl.ds(g * SCAN * 3, SCAN * 3)],
                         e3b_v.at[pl.ds(b * SCAN * 3, SCAN * 3)], sem)

    def _wait_chunk(g, b, sem):
        pltpu.make_async_copy(e3_h.at[pl.ds(g * SCAN * 3, SCAN * 3)],
                              e3b_v.at[pl.ds(b * SCAN * 3, SCAN * 3)],
                              sem).wait()

    _issue_chunk(0, 0, sem_c0)

    def _outer(gg, cur):
        for b, sem, osem in ((0, sem_c0, sem_c1), (1, sem_c1, sem_c0)):
            g = gg * 2 + b
            _wait_chunk(g, b, sem)

            @pl.when(g + 1 < NCHUNK)
            def _():
                _issue_chunk(g + 1, 1 - b, osem)

            boff = b * SCAN * 3

            def _p1(i, _):
                for u in range(UNR):
                    ii = i * UNR + u
                    d = plsc.load_gather(e3b_v, [boff + ii * (L * 3) + iota3])
                    loc = d - n0
                    m = (loc >= 0) & (loc < PER)
                    cnts_v[pl.ds(ii * L, L)] = (
                        plsc.all_reduce_population_count(m))
                return 0
            lax.fori_loop(0, GROUPS // UNR, _p1, 0)

            carry = cur
            for q in range(GG // L):
                idxs = (q * L + iota16) * L
                cv = plsc.load_gather(cnts_v, [idxs])
                csum = plsc.cumsum(cv)
                offs = carry + csum - cv
                goff_v[pl.ds(q * L, L)] = offs
                reld = goff_v[pl.ds(q * L, L)]
                carry = reld[L - 1] + cv[L - 1]
            cur = carry

            def _p2(i, _):
                for u in range(UNR):
                    ii = i * UNR + u
                    g3 = boff + ii * (L * 3)
                    d = plsc.load_gather(e3b_v, [g3 + iota3])
                    s = plsc.load_gather(e3b_v, [g3 + iota3 + 1])
                    e = plsc.load_gather(e3b_v, [g3 + iota3 + 2])
                    loc = d - n0
                    m = (loc >= 0) & (loc < PER)
                    base = goff_v[pl.ds(ii, L)][0]
                    csum = plsc.cumsum(m.astype(jnp.int32))
                    idx = jnp.where(m, base + csum - 1, CAP + iota16)
                    plsc.store_scatter(cl_v, [idx], loc)
                    plsc.store_scatter(cs_v, [idx], s)
                    plsc.store_scatter(ce_v, [idx], e)
                return 0
            lax.fori_loop(0, GROUPS // UNR, _p2, 0)

            def _flush(c):
                _process(c)
                return jnp.int32(0)
            cur = lax.cond(cur >= FLUSH, _flush, lambda c: c, cur)
        return cur

    cur = lax.fori_loop(0, NCHUNK // 2, _outer, jnp.int32(0))
    _process(cur)


    def _pzero(i, _):
        pooled_v[pl.ds(i * L, L)] = zero_vec
        return 0
    lax.fori_loop(0, B * (D // L), _pzero, 0)

    def _row(r, _):
        g = gid_v[pl.ds(r, L)][0]
        goff = g * D
        roff = r * L
        for jb in range(D // L):
            a = aggs[jb][pl.ds(roff, L)]
            a = jnp.where(a == NEG, zero_vec, a)
            po = pooled_v[pl.ds(goff + jb * L, L)]
            pooled_v[pl.ds(goff + jb * L, L)] = po + a
        return 0
    lax.fori_loop(0, PER, _row, 0)

    pltpu.sync_copy(pooled_v, out_h.at[pl.ds(wid * B * D, B * D)])


@functools.cache
def _sc_kernel():
    return pl.kernel(
        _sc_body,
        out_type=jax.ShapeDtypeStruct((NW * B * D,), jnp.float32),
        mesh=plsc.VectorSubcoreMesh(core_axis_name="c", subcore_axis_name="s",
                                    num_cores=NC, num_subcores=NS),
        compiler_params=pltpu.CompilerParams(needs_layout_passes=False,
                                             use_tc_tiling_on_sc=False),
        scratch_types=[
            pltpu.VMEM((NPAD,), jnp.int32),
            pltpu.VMEM(((PER + 1) * L,), jnp.float32),
            pltpu.VMEM(((PER + 1) * L,), jnp.float32),
            pltpu.VMEM(((PER + 1) * L,), jnp.float32),
            pltpu.VMEM(((PER + 1) * L,), jnp.float32),
            pltpu.VMEM(((PER + 1) * L,), jnp.float32),
            pltpu.VMEM(((PER + 1) * L,), jnp.float32),
            pltpu.VMEM(((PER + 1) * L,), jnp.float32),
            pltpu.VMEM(((PER + 1) * L,), jnp.float32),
            pltpu.VMEM((2 * SCAN * 3,), jnp.int32),
            pltpu.VMEM((CAP + L,), jnp.int32),
            pltpu.VMEM((CAP + L,), jnp.int32),
            pltpu.VMEM((CAP + L,), jnp.int32),
            pltpu.VMEM((GG * L,), jnp.int32),
            pltpu.VMEM((GG + L,), jnp.int32),
            pltpu.VMEM((2, PB), jnp.int32),
            pltpu.VMEM((2, PB, D), jnp.float32),
            pltpu.VMEM((2, PB), jnp.float32),
            pltpu.VMEM((PERP + L,), jnp.int32),
            pltpu.VMEM((B * D,), jnp.float32),
            pltpu.SemaphoreType.DMA,
            pltpu.SemaphoreType.DMA,
            pltpu.SemaphoreType.DMA,
            pltpu.SemaphoreType.DMA,
        ],
    )


def _final_body(p_ref, w_ref, b_ref, o_ref):
    s = jnp.sum(p_ref[...], axis=0)
    act = jnp.maximum(s, 0.0)
    o_ref[...] = jnp.dot(act, w_ref[...],
                         preferred_element_type=jnp.float32) + b_ref[...][None, :]


def kernel(node_ids, edge_src, edge_dst, edge_ids, graph_ids, node_embedding,
           edge_w, W, b):
    C = W.shape[1]
    nid_pad = jnp.concatenate(
        [node_ids, jnp.zeros((NPAD - N,), node_ids.dtype)])
    gid_pad = jnp.concatenate(
        [graph_ids, jnp.zeros((NPAD - N,), graph_ids.dtype)]).reshape(NW, PER)
    gid_pad = jnp.pad(gid_pad, ((0, 0), (0, PERP - PER))).reshape(-1)
    ew_flat = edge_w.reshape(-1)
    e3 = jnp.stack([edge_dst, edge_src, edge_ids], axis=1).reshape(-1)
    partials = _sc_kernel()(nid_pad, e3, gid_pad, node_embedding, ew_flat)
    partials = partials.reshape(NW, B, D)
    out = pl.pallas_call(
        _final_body,
        out_shape=jax.ShapeDtypeStruct((B, C), jnp.float32),
    )(partials, W, b)
    return out


# speedup vs baseline: 1.3438x; 1.3438x over previous
import functools

import jax
import jax.numpy as jnp
from jax import lax
from jax.experimental import pallas as pl
from jax.experimental.pallas import tpu as pltpu
from jax.experimental.pallas import tpu_sc as plsc

N = 10000
E = 320000
D = 128
B = 64
NC, NS, L = 2, 16, 16
NW = NC * NS
PER = 313
NPAD = NW * PER
PERP = 320
SCAN = 2000
GROUPS = SCAN // L
NCHUNK = E // SCAN
CAP = 4096
FLUSH = CAP - SCAN
PB = 128
UNR = 5
GG = 128
NEG = float("-inf")


def _sc_body(nid_h, dst_h, src_h, eid_h, gid_h, emb_h, ew_h, out_h,
             nid_v, agg0, agg1, agg2, agg3, agg4, agg5, agg6, agg7,
             dstb_v, srcb_v, eidb_v, cl_v, cs_v, ce_v,
             cnts_v, goff_v, vid_v, hrow_v, wv_v, gid_v, pooled_v,
             sem_c0, sem_c1, sem_g0, sem_g1):
    aggs = (agg0, agg1, agg2, agg3, agg4, agg5, agg6, agg7)
    wid = lax.axis_index("c") * NS + lax.axis_index("s")
    n0 = wid * PER

    pltpu.sync_copy(nid_h, nid_v)
    pltpu.sync_copy(gid_h.at[pl.ds(wid * PERP, PERP)], gid_v.at[pl.ds(0, PERP)])

    iota16 = jnp.arange(L, dtype=jnp.int32)
    neg_vec = jnp.full((L,), NEG, jnp.float32)
    zero_vec = jnp.zeros((L,), jnp.float32)
    dummy_ivec = jnp.full((L,), PER, jnp.int32)

    def _init(i, _):
        for a in aggs:
            a[pl.ds(i * L, L)] = neg_vec
        return 0
    lax.fori_loop(0, PER + 1, _init, 0)

    def _init2(i, _):
        cl_v[pl.ds(i * L, L)] = dummy_ivec
        cs_v[pl.ds(i * L, L)] = jnp.zeros((L,), jnp.int32)
        ce_v[pl.ds(i * L, L)] = jnp.zeros((L,), jnp.int32)
        return 0
    lax.fori_loop(0, (CAP + L) // L, _init2, 0)

    def _init3(i, _):
        cnts_v[pl.ds(i * L, L)] = jnp.zeros((L,), jnp.int32)
        return 0
    lax.fori_loop(0, GG, _init3, 0)


    def _vids_into(k, slot):
        def _vg(j, _):
            sv = cs_v[pl.ds(k * PB + j * L, L)]
            vid_v[slot, pl.ds(j * L, L)] = plsc.load_gather(nid_v, [sv])
            return 0
        lax.fori_loop(0, PB // L, _vg, 0)

    def _issue_gather(k, slot, sem):
        pltpu.async_copy(emb_h.at[vid_v.at[slot]], hrow_v.at[slot], sem)
        pltpu.async_copy(ew_h.at[ce_v.at[pl.ds(k * PB, PB)]],
                         wv_v.at[slot], sem)

    def _wait_gather(k, slot, sem):
        pltpu.make_async_copy(emb_h.at[vid_v.at[slot]], hrow_v.at[slot],
                              sem).wait()
        pltpu.make_async_copy(ew_h.at[ce_v.at[pl.ds(k * PB, PB)]],
                              wv_v.at[slot], sem).wait()

    def _rmw(k, slot):
        def _r16(j, _):
            locv = cl_v[pl.ds(k * PB + j * L, L)]
            wv16 = wv_v[slot, pl.ds(j * L, L)]
            ei0 = j * L
            for l in range(L):
                lane = jnp.full((L,), l, jnp.int32)
                rb = locv.at[lane].get(mode="promise_in_bounds")
                wb = wv16.at[lane].get(mode="promise_in_bounds")
                addr = rb * L + iota16
                ei = ei0 + l
                for jb in range(D // L):
                    hv = hrow_v[slot, ei, pl.ds(jb * L, L)]
                    av = plsc.load_gather(aggs[jb], [addr])
                    plsc.store_scatter(aggs[jb], [addr],
                                       jnp.maximum(av, hv * wb))
            return 0
        lax.fori_loop(0, PB // L, _r16, 0)

    def _process(cnt):
        nb = (cnt + PB - 1) // PB

        @pl.when(nb > 0)
        def _():
            _vids_into(0, 0)
            _issue_gather(0, 0, sem_g0)

        def _pouter(kk, _):
            for b, sem, osem in ((0, sem_g0, sem_g1), (1, sem_g1, sem_g0)):
                k = kk * 2 + b

                @pl.when(k < nb)
                def _():
                    _wait_gather(k, b, sem)

                    @pl.when(k + 1 < nb)
                    def _():
                        _vids_into(k + 1, 1 - b)
                        _issue_gather(k + 1, 1 - b, osem)

                    _rmw(k, b)
            return 0
        lax.fori_loop(0, (nb + 1) // 2, _pouter, 0)


    def _issue_chunk(g, b, sem):
        off = g * SCAN
        pltpu.async_copy(dst_h.at[pl.ds(off, SCAN)], dstb_v.at[b], sem)
        pltpu.async_copy(src_h.at[pl.ds(off, SCAN)], srcb_v.at[b], sem)
        pltpu.async_copy(eid_h.at[pl.ds(off, SCAN)], eidb_v.at[b], sem)

    def _wait_chunk(g, b, sem):
        off = g * SCAN
        pltpu.make_async_copy(dst_h.at[pl.ds(off, SCAN)], dstb_v.at[b],
                              sem).wait()
        pltpu.make_async_copy(src_h.at[pl.ds(off, SCAN)], srcb_v.at[b],
                              sem).wait()
        pltpu.make_async_copy(eid_h.at[pl.ds(off, SCAN)], eidb_v.at[b],
                              sem).wait()

    _issue_chunk(0, 0, sem_c0)

    def _outer(gg, cur):
        for b, sem, osem in ((0, sem_c0, sem_c1), (1, sem_c1, sem_c0)):
            g = gg * 2 + b
            _wait_chunk(g, b, sem)

            @pl.when(g + 1 < NCHUNK)
            def _():
                _issue_chunk(g + 1, 1 - b, osem)

            def _p1(i, _):
                for u in range(UNR):
                    ii = i * UNR + u
                    d = dstb_v[b, pl.ds(ii * L, L)]
                    loc = d - n0
                    m = (loc >= 0) & (loc < PER)
                    cnts_v[pl.ds(ii * L, L)] = (
                        plsc.all_reduce_population_count(m))
                return 0
            lax.fori_loop(0, GROUPS // UNR, _p1, 0)

            carry = cur
            for q in range(GG // L):
                idxs = (q * L + iota16) * L
                cv = plsc.load_gather(cnts_v, [idxs])
                csum = plsc.cumsum(cv)
                offs = carry + csum - cv
                goff_v[pl.ds(q * L, L)] = offs
                reld = goff_v[pl.ds(q * L, L)]
                carry = reld[L - 1] + cv[L - 1]
            cur = carry

            def _p2(i, _):
                for u in range(UNR):
                    ii = i * UNR + u
                    d = dstb_v[b, pl.ds(ii * L, L)]
                    s = srcb_v[b, pl.ds(ii * L, L)]
                    e = eidb_v[b, pl.ds(ii * L, L)]
                    loc = d - n0
                    m = (loc >= 0) & (loc < PER)
                    base = goff_v[pl.ds(ii, L)][0]
                    csum = plsc.cumsum(m.astype(jnp.int32))
                    idx = jnp.where(m, base + csum - 1, CAP + iota16)
                    plsc.store_scatter(cl_v, [idx], loc)
                    plsc.store_scatter(cs_v, [idx], s)
                    plsc.store_scatter(ce_v, [idx], e)
                return 0
            lax.fori_loop(0, GROUPS // UNR, _p2, 0)

            def _flush(c):
                _process(c)
                return jnp.int32(0)
            cur = lax.cond(cur >= FLUSH, _flush, lambda c: c, cur)
        return cur

    cur = lax.fori_loop(0, NCHUNK // 2, _outer, jnp.int32(0))
    _process(cur)


    def _pzero(i, _):
        pooled_v[pl.ds(i * L, L)] = zero_vec
        return 0
    lax.fori_loop(0, B * (D // L), _pzero, 0)

    def _row(r, _):
        g = gid_v[pl.ds(r, L)][0]
        goff = g * D
        roff = r * L
        for jb in range(D // L):
            a = aggs[jb][pl.ds(roff, L)]
            a = jnp.where(a == NEG, zero_vec, a)
            po = pooled_v[pl.ds(goff + jb * L, L)]
            pooled_v[pl.ds(goff + jb * L, L)] = po + a
        return 0
    lax.fori_loop(0, PER, _row, 0)

    pltpu.sync_copy(pooled_v, out_h.at[pl.ds(wid * B * D, B * D)])


@functools.cache
def _sc_kernel():
    return pl.kernel(
        _sc_body,
        out_type=jax.ShapeDtypeStruct((NW * B * D,), jnp.float32),
        mesh=plsc.VectorSubcoreMesh(core_axis_name="c", subcore_axis_name="s",
                                    num_cores=NC, num_subcores=NS),
        compiler_params=pltpu.CompilerParams(needs_layout_passes=False,
                                             use_tc_tiling_on_sc=False),
        scratch_types=[
            pltpu.VMEM((NPAD,), jnp.int32),
            pltpu.VMEM(((PER + 1) * L,), jnp.float32),
            pltpu.VMEM(((PER + 1) * L,), jnp.float32),
            pltpu.VMEM(((PER + 1) * L,), jnp.float32),
            pltpu.VMEM(((PER + 1) * L,), jnp.float32),
            pltpu.VMEM(((PER + 1) * L,), jnp.float32),
            pltpu.VMEM(((PER + 1) * L,), jnp.float32),
            pltpu.VMEM(((PER + 1) * L,), jnp.float32),
            pltpu.VMEM(((PER + 1) * L,), jnp.float32),
            pltpu.VMEM((2, SCAN), jnp.int32),
            pltpu.VMEM((2, SCAN), jnp.int32),
            pltpu.VMEM((2, SCAN), jnp.int32),
            pltpu.VMEM((CAP + L,), jnp.int32),
            pltpu.VMEM((CAP + L,), jnp.int32),
            pltpu.VMEM((CAP + L,), jnp.int32),
            pltpu.VMEM((GG * L,), jnp.int32),
            pltpu.VMEM((GG + L,), jnp.int32),
            pltpu.VMEM((2, PB), jnp.int32),
            pltpu.VMEM((2, PB, D), jnp.float32),
            pltpu.VMEM((2, PB), jnp.float32),
            pltpu.VMEM((PERP + L,), jnp.int32),
            pltpu.VMEM((B * D,), jnp.float32),
            pltpu.SemaphoreType.DMA,
            pltpu.SemaphoreType.DMA,
            pltpu.SemaphoreType.DMA,
            pltpu.SemaphoreType.DMA,
        ],
    )


def _final_body(p_ref, w_ref, b_ref, o_ref):
    s = jnp.sum(p_ref[...], axis=0)
    act = jnp.maximum(s, 0.0)
    o_ref[...] = jnp.dot(act, w_ref[...],
                         preferred_element_type=jnp.float32) + b_ref[...][None, :]


def kernel(node_ids, edge_src, edge_dst, edge_ids, graph_ids, node_embedding,
           edge_w, W, b):
    C = W.shape[1]
    nid_pad = jnp.concatenate(
        [node_ids, jnp.zeros((NPAD - N,), node_ids.dtype)])
    gid_pad = jnp.concatenate(
        [graph_ids, jnp.zeros((NPAD - N,), graph_ids.dtype)]).reshape(NW, PER)
    gid_pad = jnp.pad(gid_pad, ((0, 0), (0, PERP - PER))).reshape(-1)
    ew_flat = edge_w.reshape(-1)
    partials = _sc_kernel()(nid_pad, edge_dst, edge_src, edge_ids, gid_pad,
                            node_embedding, ew_flat)
    partials = partials.reshape(NW, B, D)
    out = pl.pallas_call(
        _final_body,
        out_shape=jax.ShapeDtypeStruct((B, C), jnp.float32),
    )(partials, W, b)
    return out
